# baseline (speedup 1.0000x reference)
import functools

import jax
import jax.numpy as jnp
from jax import lax
from jax.experimental import pallas as pl
from jax.experimental.pallas import tpu as pltpu
from jax.experimental.pallas import tpu_sc as plsc

DIM = 64
PDIM = 128
NCHUNK = 5
NC, NS = 2, 16
NW = NC * NS


@functools.lru_cache(maxsize=None)
def _make_gather(n_batch, seq, n_table_rows):
    SEQ = seq
    GB = 4
    HB = GB // 2
    b_per_w = n_batch // NW
    n_groups = b_per_w // GB
    idx_per_w = b_per_w * SEQ
    grows = GB * SEQ
    assert b_per_w * NW == n_batch
    assert n_groups * GB == b_per_w
    assert n_groups % 2 == 0 and grows % 8 == 0

    mesh = plsc.VectorSubcoreMesh(core_axis_name="c", subcore_axis_name="s")

    @functools.partial(
        pl.kernel,
        out_type=jax.ShapeDtypeStruct((n_batch, SEQ, DIM), jnp.float32),
        mesh=mesh,
        scratch_types=[
            pltpu.VMEM((idx_per_w,), jnp.int32),
            pltpu.VMEM((2, grows, PDIM), jnp.float32),
            pltpu.VMEM((2, HB, SEQ, DIM), jnp.float32),
            pltpu.SemaphoreType.DMA,
            pltpu.SemaphoreType.DMA,
            pltpu.SemaphoreType.DMA,
            pltpu.SemaphoreType.DMA,
        ],
    )
    def gather_kernel(table_hbm, idx_hbm, out_hbm, idx_v, rows_v, pack_v,
                      gsem0, gsem1, psem0, psem1):
        gsems = (gsem0, gsem1)
        psems = (psem0, psem1)
        wid = lax.axis_index("s") * NC + lax.axis_index("c")
        b0 = wid * b_per_w

        pltpu.sync_copy(idx_hbm.at[pl.ds(wid * idx_per_w, idx_per_w)], idx_v)

        def gather(g, buf):
            pltpu.async_copy(
                table_hbm.at[idx_v.at[pl.ds(g * grows, grows)]],
                rows_v.at[buf], gsems[buf])

        def wait_gather(buf):
            pltpu.make_async_copy(
                table_hbm.at[pl.ds(0, grows)], rows_v.at[buf],
                gsems[buf]).wait()

        def repack(buf, h):
            for i in range(HB):
                base = (h * HB + i) * SEQ

                @pl.loop(0, SEQ, unroll=2)
                def _(s):
                    for c in range(DIM // 16):
                        pack_v[h, i, s, pl.ds(c * 16, 16)] = (
                            rows_v[buf, base + s, pl.ds(c * 16, 16)])

        def store(g, h):
            pltpu.async_copy(
                pack_v.at[h],
                out_hbm.at[pl.ds(b0 + g * GB + h * HB, HB)],
                psems[h])

        def wait_store(h):
            pltpu.make_async_copy(
                pack_v.at[h], out_hbm.at[pl.ds(b0, HB)], psems[h]).wait()

        gather(0, 0)
        gather(1, 1)

        @pl.loop(0, n_groups, step=2)
        def _(j0):
            for buf in range(2):
                j = j0 + buf
                wait_gather(buf)
                for h in range(2):

                    @pl.when(j > 0)
                    def _():
                        wait_store(h)

                    repack(buf, h)
                    store(j, h)

                @pl.when(j < n_groups - 2)
                def _():
                    gather(j + 2, buf)

        wait_store(0)
        wait_store(1)

    return gather_kernel


def kernel(node_positions, psne_layer):
    b, s = node_positions.shape
    table128 = jnp.pad(psne_layer, ((0, 0), (0, PDIM - DIM)))
    sc = s // NCHUNK
    fn = _make_gather(b, sc, psne_layer.shape[0])
    outs = []
    for k in range(NCHUNK):
        idx_k = node_positions[:, k * sc:(k + 1) * sc]
        outs.append(fn(table128, idx_k.reshape(b * sc).astype(jnp.int32)))
    return jnp.concatenate(outs, axis=1)


# speedup vs baseline: 1.0572x; 1.0572x over previous
import functools

import jax
import jax.numpy as jnp
from jax import lax
from jax.experimental import pallas as pl
from jax.experimental.pallas import tpu as pltpu
from jax.experimental.pallas import tpu_sc as plsc

DIM = 64
PDIM = 128
NCHUNK = 2
NC, NS = 2, 16
NW = NC * NS


@functools.lru_cache(maxsize=None)
def _make_gather(n_batch, seq, n_table_rows):
    SEQ = seq
    GB = 8 if seq % 8 else 4
    HB = GB // 2
    b_per_w = n_batch // NW
    n_groups = b_per_w // GB
    idx_per_w = b_per_w * SEQ
    grows = GB * SEQ
    assert b_per_w * NW == n_batch
    assert n_groups * GB == b_per_w
    assert n_groups % 2 == 0 and grows % 8 == 0

    mesh = plsc.VectorSubcoreMesh(core_axis_name="c", subcore_axis_name="s")

    @functools.partial(
        pl.kernel,
        out_type=jax.ShapeDtypeStruct((n_batch, SEQ, DIM), jnp.float32),
        mesh=mesh,
        scratch_types=[
            pltpu.VMEM((idx_per_w,), jnp.int32),
            pltpu.VMEM((2, grows, PDIM), jnp.float32),
            pltpu.VMEM((2, HB, SEQ, DIM), jnp.float32),
            pltpu.SemaphoreType.DMA,
            pltpu.SemaphoreType.DMA,
            pltpu.SemaphoreType.DMA,
            pltpu.SemaphoreType.DMA,
        ],
    )
    def gather_kernel(table_hbm, idx_hbm, out_hbm, idx_v, rows_v, pack_v,
                      gsem0, gsem1, psem0, psem1):
        gsems = (gsem0, gsem1)
        psems = (psem0, psem1)
        wid = lax.axis_index("s") * NC + lax.axis_index("c")
        b0 = wid * b_per_w

        pltpu.sync_copy(idx_hbm.at[pl.ds(wid * idx_per_w, idx_per_w)], idx_v)

        def gather(g, buf):
            pltpu.async_copy(
                table_hbm.at[idx_v.at[pl.ds(g * grows, grows)]],
                rows_v.at[buf], gsems[buf])

        def wait_gather(buf):
            pltpu.make_async_copy(
                table_hbm.at[pl.ds(0, grows)], rows_v.at[buf],
                gsems[buf]).wait()

        def repack(buf, h):
            for i in range(HB):
                base = (h * HB + i) * SEQ

                @pl.loop(0, SEQ, unroll=2)
                def _(s):
                    for c in range(DIM // 16):
                        pack_v[h, i, s, pl.ds(c * 16, 16)] = (
                            rows_v[buf, base + s, pl.ds(c * 16, 16)])

        def store(g, h):
            pltpu.async_copy(
                pack_v.at[h],
                out_hbm.at[pl.ds(b0 + g * GB + h * HB, HB)],
                psems[h])

        def wait_store(h):
            pltpu.make_async_copy(
                pack_v.at[h], out_hbm.at[pl.ds(b0, HB)], psems[h]).wait()

        gather(0, 0)
        gather(1, 1)

        @pl.loop(0, n_groups, step=2)
        def _(j0):
            for buf in range(2):
                j = j0 + buf
                wait_gather(buf)
                for h in range(2):

                    @pl.when(j > 0)
                    def _():
                        wait_store(h)

                    repack(buf, h)
                    store(j, h)

                @pl.when(j < n_groups - 2)
                def _():
                    gather(j + 2, buf)

        wait_store(0)
        wait_store(1)

    return gather_kernel


def kernel(node_positions, psne_layer):
    b, s = node_positions.shape
    table128 = jnp.pad(psne_layer, ((0, 0), (0, PDIM - DIM)))
    sc = s // NCHUNK
    fn = _make_gather(b, sc, psne_layer.shape[0])
    outs = []
    for k in range(NCHUNK):
        idx_k = node_positions[:, k * sc:(k + 1) * sc]
        outs.append(fn(table128, idx_k.reshape(b * sc).astype(jnp.int32)))
    return jnp.concatenate(outs, axis=1)


# speedup vs baseline: 1.2709x; 1.2021x over previous
import functools

import jax
import jax.numpy as jnp
from jax import lax
from jax.experimental import pallas as pl
from jax.experimental.pallas import tpu as pltpu
from jax.experimental.pallas import tpu_sc as plsc

DIM = 64
PDIM = 128
SEQ = 50
GB = 4
HB = GB // 2
NC, NS = 2, 16
NW = NC * NS


@functools.lru_cache(maxsize=None)
def _make_gather(n_batch, n_table_rows):
    b_per_w = n_batch // NW
    n_groups = b_per_w // GB
    idx_per_w = b_per_w * SEQ
    grows = GB * SEQ
    assert b_per_w * NW == n_batch
    assert n_groups * GB == b_per_w
    assert n_groups % 2 == 0 and grows % 8 == 0

    mesh = plsc.VectorSubcoreMesh(core_axis_name="c", subcore_axis_name="s")

    @functools.partial(
        pl.kernel,
        out_type=jax.ShapeDtypeStruct((n_batch, SEQ, DIM), jnp.float32),
        mesh=mesh,
        scratch_types=[
            pltpu.VMEM((idx_per_w,), jnp.int32),
            pltpu.VMEM((2, grows, PDIM), jnp.float32),
            pltpu.VMEM((2, HB, SEQ, DIM), jnp.float32),
            pltpu.SemaphoreType.DMA,
            pltpu.SemaphoreType.DMA,
            pltpu.SemaphoreType.DMA,
            pltpu.SemaphoreType.DMA,
        ],
    )
    def gather_kernel(table_hbm, idx_hbm, out_hbm, idx_v, rows_v, pack_v,
                      gsem0, gsem1, psem0, psem1):
        gsems = (gsem0, gsem1)
        psems = (psem0, psem1)
        wid = lax.axis_index("s") * NC + lax.axis_index("c")
        b0 = wid * b_per_w

        pltpu.sync_copy(idx_hbm.at[pl.ds(wid * idx_per_w, idx_per_w)], idx_v)

        def gather(g, buf):
            pltpu.async_copy(
                table_hbm.at[idx_v.at[pl.ds(g * grows, grows)]],
                rows_v.at[buf], gsems[buf])

        def wait_gather(buf):
            pltpu.make_async_copy(
                table_hbm.at[pl.ds(0, grows)], rows_v.at[buf],
                gsems[buf]).wait()

        def repack(buf, h):
            for i in range(HB):
                base = (h * HB + i) * SEQ

                @pl.loop(0, SEQ, unroll=2)
                def _(s):
                    for c in range(DIM // 16):
                        pack_v[h, i, s, pl.ds(c * 16, 16)] = (
                            rows_v[buf, base + s, pl.ds(c * 16, 16)])

        def store(g, h):
            pltpu.async_copy(
                pack_v.at[h],
                out_hbm.at[pl.ds(b0 + g * GB + h * HB, HB)],
                psems[h])

        def wait_store(h):
            pltpu.make_async_copy(
                pack_v.at[h], out_hbm.at[pl.ds(b0, HB)], psems[h]).wait()

        gather(0, 0)
        gather(1, 1)

        @pl.loop(0, n_groups, step=2)
        def _(j0):
            for buf in range(2):
                j = j0 + buf
                wait_gather(buf)
                for h in range(2):

                    @pl.when(j > 0)
                    def _():
                        wait_store(h)

                    repack(buf, h)
                    store(j, h)

                @pl.when(j < n_groups - 2)
                def _():
                    gather(j + 2, buf)

        wait_store(0)
        wait_store(1)

    return gather_kernel


def kernel(node_positions, psne_layer):
    b, s = node_positions.shape
    idx_flat = node_positions.reshape(b * s).astype(jnp.int32)
    table128 = jnp.pad(psne_layer, ((0, 0), (0, PDIM - DIM)))
    fn = _make_gather(b, psne_layer.shape[0])
    return fn(table128, idx_flat)
